# Initial kernel scaffold; baseline (speedup 1.0000x reference)
import jax
import jax.numpy as jnp
from jax.experimental import pallas as pl


def kernel(x, edge_index, params):
    raise NotImplementedError("write your pallas kernel here")


# speedup vs baseline: 4.2762x; 4.2762x over previous
import functools

import jax
import jax.numpy as jnp
from jax import lax
from jax.experimental import pallas as pl
from jax.experimental.pallas import tpu as pltpu
from jax.experimental.pallas import tpu_sc as plsc

_N = 10000
_E = 320000
_HID = 256

_CHUNK = 128
_TILES = 16
_CORES = 2
_E_PAD = 323584
_NCHUNK_TOTAL = _E_PAD // _CHUNK
_NCHUNK_TILE = _NCHUNK_TOTAL // _TILES
_ACC_ROWS = 10240
_ROWS_TILE = _ACC_ROWS // _TILES
_ZROWS = 128
_TAB_PAD = 8
_DH = 64


def _agg_body(tab_l, tab_r, src_hbm, dst_hbm, out_hbm,
              idx_s, idx_d, rows, zbuf, acc, sem0, sem1):
    c = lax.axis_index("c")
    s = lax.axis_index("s")
    dh = rows.shape[2]

    def _zrow(r, carry):
        for j in range(dh // 16):
            zbuf[r, pl.ds(j * 16, 16)] = jnp.zeros((16,), jnp.float32)
        return carry
    lax.fori_loop(0, _ZROWS, _zrow, 0)
    for k in range(_ROWS_TILE // _ZROWS):
        pltpu.sync_copy(zbuf, acc.at[pl.ds(s * _ROWS_TILE + k * _ZROWS, _ZROWS)])
    plsc.subcore_barrier()

    pltpu.sync_copy(src_hbm.at[s], idx_s)
    pltpu.sync_copy(dst_hbm.at[s], idx_d)

    sems = [sem0, sem1]

    def _run(tab):
        for b in range(2):
            pltpu.async_copy(tab.at[idx_s.at[b]], rows.at[b], sems[b])

        def _pair(t, carry):
            for b in range(2):
                g = t * 2 + b
                pltpu.make_async_copy(
                    tab.at[pl.ds(0, _CHUNK)], rows.at[b], sems[b]).wait()
                pltpu.sync_copy(rows.at[b], acc.at[idx_d.at[g]], add=True)

                @pl.when(g + 2 < _NCHUNK_TILE)
                def _():
                    pltpu.async_copy(tab.at[idx_s.at[g + 2]], rows.at[b], sems[b])
            return carry
        lax.fori_loop(0, _NCHUNK_TILE // 2, _pair, 0)

    @pl.when(c == 0)
    def _():
        _run(tab_l)

    @pl.when(c == 1)
    def _():
        _run(tab_r)

    plsc.subcore_barrier()
    pltpu.sync_copy(acc.at[pl.ds(s * _ROWS_TILE, _ROWS_TILE)],
                    out_hbm.at[c, pl.ds(s * _ROWS_TILE, _ROWS_TILE)])


@functools.lru_cache(maxsize=None)
def _make_agg():
    mesh = plsc.VectorSubcoreMesh(core_axis_name="c", subcore_axis_name="s",
                                  num_cores=_CORES, num_subcores=_TILES)
    return pl.kernel(
        _agg_body,
        out_type=jax.ShapeDtypeStruct((_CORES, _ACC_ROWS, _DH), jnp.float32),
        mesh=mesh,
        scratch_types=[
            pltpu.VMEM((_NCHUNK_TILE, _CHUNK), jnp.int32),
            pltpu.VMEM((_NCHUNK_TILE, _CHUNK), jnp.int32),
            pltpu.VMEM((2, _CHUNK, _DH), jnp.float32),
            pltpu.VMEM((_ZROWS, _DH), jnp.float32),
            pltpu.VMEM_SHARED((_ACC_ROWS, _DH), jnp.float32),
            pltpu.SemaphoreType.DMA,
            pltpu.SemaphoreType.DMA,
        ],
        compiler_params=pltpu.CompilerParams(use_tc_tiling_on_sc=False),
    )


def _mlp_stats_body(*refs):
    h_ref = refs[0]
    n_pieces = h_ref.shape[1] // _DH
    aggs = refs[1:1 + n_pieces]
    w1, b1, w2, b2, m_ref, st_ref = refs[1 + n_pieces:]
    i = pl.program_id(0)
    x = h_ref[...] + jnp.concatenate([a[...] for a in aggs], axis=1)
    t = jnp.maximum(
        jnp.dot(x, w1[...], preferred_element_type=jnp.float32,
                precision=lax.Precision.DEFAULT) + b1[...], 0.0)
    m = jnp.dot(t, w2[...], preferred_element_type=jnp.float32,
                precision=lax.Precision.DEFAULT) + b2[...]
    m_ref[...] = m
    sums = jnp.sum(m, axis=0, keepdims=True)

    @pl.when(i == 0)
    def _():
        st_ref[0:1, :] = sums

    @pl.when(i > 0)
    def _():
        st_ref[0:1, :] = st_ref[0:1, :] + sums


def _bn_adapter_body(m_ref, st_ref, g_ref, b_ref, a1, ab1, a2, ab2, o_ref,
                     sdev, *, relu):
    ph = pl.program_id(0)
    i = pl.program_id(1)
    n_f = jnp.float32(_N)
    mean = st_ref[0:1, :] / n_f
    c = m_ref[...] - mean

    @pl.when(jnp.logical_and(ph == 0, i == 0))
    def _():
        sdev[...] = jnp.zeros_like(sdev)

    @pl.when(ph == 0)
    def _():
        sdev[0:1, :] = sdev[0:1, :] + jnp.sum(c * c, axis=0, keepdims=True)

    @pl.when(ph == 1)
    def _():
        var = sdev[0:1, :] / n_f
        mh = c / jnp.sqrt(var + 1e-5) * g_ref[...] + b_ref[...]
        a = jnp.maximum(
            jnp.dot(mh, a1[...], preferred_element_type=jnp.float32,
                    precision=lax.Precision.DEFAULT) + ab1[...], 0.0)
        o = mh + jnp.dot(a, a2[...], preferred_element_type=jnp.float32,
                    precision=lax.Precision.DEFAULT) + ab2[...]
        if relu:
            o = jnp.maximum(o, 0.0)
        o_ref[...] = o


_BLK = 1000
_GRID = _N // _BLK


@functools.lru_cache(maxsize=None)
def _make_mlp_stats(d):
    n_pieces = d // _DH
    return pl.pallas_call(
        _mlp_stats_body,
        grid=(_GRID,),
        in_specs=[
            pl.BlockSpec((_BLK, d), lambda i: (i, 0)),
        ] + [
            pl.BlockSpec((_BLK, _DH), lambda i: (i, 0))
            for _ in range(n_pieces)
        ] + [
            pl.BlockSpec((d, _HID), lambda i: (0, 0)),
            pl.BlockSpec((1, _HID), lambda i: (0, 0)),
            pl.BlockSpec((_HID, _HID), lambda i: (0, 0)),
            pl.BlockSpec((1, _HID), lambda i: (0, 0)),
        ],
        out_specs=[
            pl.BlockSpec((_BLK, _HID), lambda i: (i, 0)),
            pl.BlockSpec((8, _HID), lambda i: (0, 0)),
        ],
        out_shape=[
            jax.ShapeDtypeStruct((_N, _HID), jnp.float32),
            jax.ShapeDtypeStruct((8, _HID), jnp.float32),
        ],
    )


@functools.lru_cache(maxsize=None)
def _make_bn_adapter(relu):
    return pl.pallas_call(
        functools.partial(_bn_adapter_body, relu=relu),
        grid=(2, _GRID),
        scratch_shapes=[pltpu.VMEM((8, _HID), jnp.float32)],
        in_specs=[
            pl.BlockSpec((_BLK, _HID), lambda p, i: (i, 0)),
            pl.BlockSpec((8, _HID), lambda p, i: (0, 0)),
            pl.BlockSpec((1, _HID), lambda p, i: (0, 0)),
            pl.BlockSpec((1, _HID), lambda p, i: (0, 0)),
            pl.BlockSpec((_HID, 64), lambda p, i: (0, 0)),
            pl.BlockSpec((1, 64), lambda p, i: (0, 0)),
            pl.BlockSpec((64, _HID), lambda p, i: (0, 0)),
            pl.BlockSpec((1, _HID), lambda p, i: (0, 0)),
        ],
        out_specs=pl.BlockSpec((_BLK, _HID), lambda p, i: (i, 0)),
        out_shape=jax.ShapeDtypeStruct((_N, _HID), jnp.float32),
    )


def kernel(x, edge_index, params):
    src = edge_index[0]
    dst = edge_index[1]
    pad = _E_PAD - _E
    src_p = jnp.concatenate(
        [src, jnp.full((pad,), _N, jnp.int32)]).reshape(
            _TILES, _NCHUNK_TILE, _CHUNK)
    dst_p = jnp.concatenate(
        [dst, jnp.zeros((pad,), jnp.int32)]).reshape(
            _TILES, _NCHUNK_TILE, _CHUNK)

    h = x
    n_layers = 3
    zpad = jnp.zeros((_TAB_PAD, _DH), jnp.float32)
    for i in range(n_layers):
        d = h.shape[1]
        n_pieces = d // _DH
        tabs = [jnp.concatenate([h[:, q * _DH:(q + 1) * _DH], zpad], axis=0)
                for q in range(n_pieces)]
        aggs = []
        for q in range(0, n_pieces, 2):
            agg2 = _make_agg()(tabs[q], tabs[q + 1], src_p, dst_p)
            aggs += [agg2[0], agg2[1]]
        m, st = _make_mlp_stats(d)(
            h, *aggs,
            params['lin1_W_%d' % i], params['lin1_b_%d' % i].reshape(1, _HID),
            params['lin2_W_%d' % i], params['lin2_b_%d' % i].reshape(1, _HID))
        h = _make_bn_adapter(i < n_layers - 1)(
            m, st,
            params['bn_g_%d' % i].reshape(1, _HID),
            params['bn_b_%d' % i].reshape(1, _HID),
            params['ad1_W_%d' % i], params['ad1_b_%d' % i].reshape(1, 64),
            params['ad2_W_%d' % i], params['ad2_b_%d' % i].reshape(1, _HID))
    return h


# speedup vs baseline: 6.5098x; 1.5223x over previous
import functools

import jax
import jax.numpy as jnp
from jax import lax
from jax.experimental import pallas as pl
from jax.experimental.pallas import tpu as pltpu
from jax.experimental.pallas import tpu_sc as plsc

_N = 10000
_E = 320000
_HID = 256

_CHUNK = 128
_TILES = 16
_CORES = 2
_E_PAD = 327680
_NCHUNK_TOTAL = _E_PAD // _CHUNK
_NCHUNK_TILE = _NCHUNK_TOTAL // _TILES
_ACC_ROWS = 10240
_ROWS_TILE = _ACC_ROWS // _TILES
_ZROWS = 128
_TAB_PAD = 64
_DH = 64


def _agg_body(tab_l, tab_r, src_hbm, dst_hbm, out_hbm,
              idx_s, idx_d, rows, zbuf, acc,
              sg0, sg1, sg2, sg3, ss0, ss1, ss2, ss3):
    c = lax.axis_index("c")
    s = lax.axis_index("s")
    dh = rows.shape[2]
    sg = [sg0, sg1, sg2, sg3]
    ss = [ss0, ss1, ss2, ss3]

    def _zrow(r, carry):
        for j in range(dh // 16):
            zbuf[r, pl.ds(j * 16, 16)] = jnp.zeros((16,), jnp.float32)
        return carry
    lax.fori_loop(0, _ZROWS, _zrow, 0)
    for k in range(_ROWS_TILE // _ZROWS):
        pltpu.sync_copy(zbuf, acc.at[pl.ds(s * _ROWS_TILE + k * _ZROWS, _ZROWS)])
    plsc.subcore_barrier()

    pltpu.sync_copy(src_hbm.at[s], idx_s)
    pltpu.sync_copy(dst_hbm.at[s], idx_d)

    def _run(tab):
        def _wait_gather(slot):
            pltpu.make_async_copy(
                tab.at[pl.ds(0, _CHUNK)], rows.at[slot], sg[slot]).wait()

        def _wait_scatter(slot):
            pltpu.make_async_copy(
                tab.at[pl.ds(0, _CHUNK)], rows.at[slot], ss[slot]).wait()

        for j in range(2):
            pltpu.async_copy(tab.at[idx_s.at[j]], rows.at[j], sg[j])

        def _dsuper(u, carry):
            for half in range(2):
                bank = 2 * half
                other = 2 - bank
                for j in range(2):
                    g = (2 * u + half) * 2 + j
                    _wait_gather(bank + j)
                    pltpu.async_copy(
                        rows.at[bank + j], acc.at[idx_d.at[g]], ss[bank + j],
                        add=True)
                for j in range(2):
                    gn = (2 * u + half + 1) * 2 + j
                    if half == 0:
                        @pl.when(u >= 1)
                        def _():
                            _wait_scatter(other + j)
                        pltpu.async_copy(
                            tab.at[idx_s.at[gn]], rows.at[other + j],
                            sg[other + j])
                    else:
                        _wait_scatter(other + j)

                        @pl.when(u < _NCHUNK_TILE // 4 - 1)
                        def _():
                            pltpu.async_copy(
                                tab.at[idx_s.at[gn]], rows.at[other + j],
                                sg[other + j])
            return carry
        lax.fori_loop(0, _NCHUNK_TILE // 4, _dsuper, 0)
        for j in range(2):
            _wait_scatter(2 + j)

    @pl.when(c == 0)
    def _():
        _run(tab_l)

    @pl.when(c == 1)
    def _():
        _run(tab_r)

    plsc.subcore_barrier()
    pltpu.sync_copy(acc.at[pl.ds(s * _ROWS_TILE, _ROWS_TILE)],
                    out_hbm.at[c, pl.ds(s * _ROWS_TILE, _ROWS_TILE)])


@functools.lru_cache(maxsize=None)
def _make_agg():
    mesh = plsc.VectorSubcoreMesh(core_axis_name="c", subcore_axis_name="s",
                                  num_cores=_CORES, num_subcores=_TILES)
    return pl.kernel(
        _agg_body,
        out_type=jax.ShapeDtypeStruct((_CORES, _ACC_ROWS, _DH), jnp.float32),
        mesh=mesh,
        scratch_types=[
            pltpu.VMEM((_NCHUNK_TILE, _CHUNK), jnp.int32),
            pltpu.VMEM((_NCHUNK_TILE, _CHUNK), jnp.int32),
            pltpu.VMEM((4, _CHUNK, _DH), jnp.float32),
            pltpu.VMEM((_ZROWS, _DH), jnp.float32),
            pltpu.VMEM_SHARED((_ACC_ROWS, _DH), jnp.float32),
        ] + [pltpu.SemaphoreType.DMA] * 8,
        compiler_params=pltpu.CompilerParams(use_tc_tiling_on_sc=False),
    )


def _mlp_stats_body(*refs):
    h_ref = refs[0]
    n_pieces = h_ref.shape[1] // _DH
    aggs = refs[1:1 + n_pieces]
    w1, b1, w2, b2, m_ref, st_ref = refs[1 + n_pieces:]
    i = pl.program_id(0)
    x = h_ref[...] + jnp.concatenate([a[...] for a in aggs], axis=1)
    t = jnp.maximum(
        jnp.dot(x, w1[...], preferred_element_type=jnp.float32,
                precision=lax.Precision.DEFAULT) + b1[...], 0.0)
    m = jnp.dot(t, w2[...], preferred_element_type=jnp.float32,
                precision=lax.Precision.DEFAULT) + b2[...]
    m_ref[...] = m
    sums = jnp.sum(m, axis=0, keepdims=True)

    @pl.when(i == 0)
    def _():
        st_ref[0:1, :] = sums

    @pl.when(i > 0)
    def _():
        st_ref[0:1, :] = st_ref[0:1, :] + sums


def _bn_adapter_body(m_ref, st_ref, g_ref, b_ref, a1, ab1, a2, ab2, o_ref,
                     sdev, *, relu):
    ph = pl.program_id(0)
    i = pl.program_id(1)
    n_f = jnp.float32(_N)
    mean = st_ref[0:1, :] / n_f
    c = m_ref[...] - mean

    @pl.when(jnp.logical_and(ph == 0, i == 0))
    def _():
        sdev[...] = jnp.zeros_like(sdev)

    @pl.when(ph == 0)
    def _():
        sdev[0:1, :] = sdev[0:1, :] + jnp.sum(c * c, axis=0, keepdims=True)

    @pl.when(ph == 1)
    def _():
        var = sdev[0:1, :] / n_f
        mh = c / jnp.sqrt(var + 1e-5) * g_ref[...] + b_ref[...]
        a = jnp.maximum(
            jnp.dot(mh, a1[...], preferred_element_type=jnp.float32,
                    precision=lax.Precision.DEFAULT) + ab1[...], 0.0)
        o = mh + jnp.dot(a, a2[...], preferred_element_type=jnp.float32,
                    precision=lax.Precision.DEFAULT) + ab2[...]
        if relu:
            o = jnp.maximum(o, 0.0)
        o_ref[...] = o


_BLK = 1000
_GRID = _N // _BLK


@functools.lru_cache(maxsize=None)
def _make_mlp_stats(d):
    n_pieces = d // _DH
    return pl.pallas_call(
        _mlp_stats_body,
        grid=(_GRID,),
        in_specs=[
            pl.BlockSpec((_BLK, d), lambda i: (i, 0)),
        ] + [
            pl.BlockSpec((_BLK, _DH), lambda i: (i, 0))
            for _ in range(n_pieces)
        ] + [
            pl.BlockSpec((d, _HID), lambda i: (0, 0)),
            pl.BlockSpec((1, _HID), lambda i: (0, 0)),
            pl.BlockSpec((_HID, _HID), lambda i: (0, 0)),
            pl.BlockSpec((1, _HID), lambda i: (0, 0)),
        ],
        out_specs=[
            pl.BlockSpec((_BLK, _HID), lambda i: (i, 0)),
            pl.BlockSpec((8, _HID), lambda i: (0, 0)),
        ],
        out_shape=[
            jax.ShapeDtypeStruct((_N, _HID), jnp.float32),
            jax.ShapeDtypeStruct((8, _HID), jnp.float32),
        ],
    )


@functools.lru_cache(maxsize=None)
def _make_bn_adapter(relu):
    return pl.pallas_call(
        functools.partial(_bn_adapter_body, relu=relu),
        grid=(2, _GRID),
        scratch_shapes=[pltpu.VMEM((8, _HID), jnp.float32)],
        in_specs=[
            pl.BlockSpec((_BLK, _HID), lambda p, i: (i, 0)),
            pl.BlockSpec((8, _HID), lambda p, i: (0, 0)),
            pl.BlockSpec((1, _HID), lambda p, i: (0, 0)),
            pl.BlockSpec((1, _HID), lambda p, i: (0, 0)),
            pl.BlockSpec((_HID, 64), lambda p, i: (0, 0)),
            pl.BlockSpec((1, 64), lambda p, i: (0, 0)),
            pl.BlockSpec((64, _HID), lambda p, i: (0, 0)),
            pl.BlockSpec((1, _HID), lambda p, i: (0, 0)),
        ],
        out_specs=pl.BlockSpec((_BLK, _HID), lambda p, i: (i, 0)),
        out_shape=jax.ShapeDtypeStruct((_N, _HID), jnp.float32),
    )


def kernel(x, edge_index, params):
    src = edge_index[0]
    dst = edge_index[1]
    pad = _E_PAD - _E
    ar = jnp.arange(pad, dtype=jnp.int32)
    src_p = jnp.concatenate(
        [src, _N + ar % _TAB_PAD]).reshape(_TILES, _NCHUNK_TILE, _CHUNK)
    dst_p = jnp.concatenate(
        [dst, _N + ar % (_ACC_ROWS - _N)]).reshape(
            _TILES, _NCHUNK_TILE, _CHUNK)

    h = x
    n_layers = 3
    zpad = jnp.zeros((_TAB_PAD, _DH), jnp.float32)
    for i in range(n_layers):
        d = h.shape[1]
        n_pieces = d // _DH
        tabs = [jnp.concatenate([h[:, q * _DH:(q + 1) * _DH], zpad], axis=0)
                for q in range(n_pieces)]
        aggs = []
        for q in range(0, n_pieces, 2):
            agg2 = _make_agg()(tabs[q], tabs[q + 1], src_p, dst_p)
            aggs += [agg2[0], agg2[1]]
        m, st = _make_mlp_stats(d)(
            h, *aggs,
            params['lin1_W_%d' % i], params['lin1_b_%d' % i].reshape(1, _HID),
            params['lin2_W_%d' % i], params['lin2_b_%d' % i].reshape(1, _HID))
        h = _make_bn_adapter(i < n_layers - 1)(
            m, st,
            params['bn_g_%d' % i].reshape(1, _HID),
            params['bn_b_%d' % i].reshape(1, _HID),
            params['ad1_W_%d' % i], params['ad1_b_%d' % i].reshape(1, 64),
            params['ad2_W_%d' % i], params['ad2_b_%d' % i].reshape(1, _HID))
    return h
